# baseline (speedup 1.0000x reference)
import jax
import jax.numpy as jnp
from jax.experimental import pallas as pl
from jax.experimental.pallas import tpu as pltpu

BM = 1024
BN = 512


def _quant_matmul_kernel(b_ref, c_ref, x_ref, pi_ref, idx_ref, yhat_ref,
                         ybuf0_ref, ybuf1_ref):
    t = pl.program_id(0)

    def _phase(dot_ref, quant_ref):
        dot_ref[...] = jnp.dot(
            x_ref[...], pi_ref[...], preferred_element_type=jnp.float32)
        y = quant_ref[...]
        u = jnp.zeros(y.shape, jnp.float32)
        for k in range(b_ref.shape[0]):
            u = u + jnp.where(y > b_ref[k],
                              (c_ref[k + 1] - c_ref[k]) + 16.0, 0.0)
        idx_f = jnp.floor(u * (1.0 / 16.0))
        idx_ref[...] = idx_f.astype(jnp.int32)
        yhat_ref[...] = ((u - 16.0 * idx_f) + c_ref[0]).astype(jnp.bfloat16)

    @pl.when(t % 2 == 0)
    def _():
        _phase(ybuf0_ref, ybuf1_ref)

    @pl.when(t % 2 == 1)
    def _():
        _phase(ybuf1_ref, ybuf0_ref)


def _unrotate_kernel(yhat_ref, pi_ref, out_ref):
    out_ref[...] = jax.lax.dot_general(
        yhat_ref[...], pi_ref[...],
        dimension_numbers=(((1,), (1,)), ((), ())),
        preferred_element_type=jnp.float32,
    )


def kernel(x, Pi, centroids, boundaries):
    M, d = x.shape
    nm, nn = M // BM, d // BN

    x_bf = x.astype(jnp.bfloat16)
    pi_bf = Pi.astype(jnp.bfloat16)

    last = nm * nn - 1

    idx, yhat = pl.pallas_call(
        _quant_matmul_kernel,
        grid=(nm * nn + 1,),
        in_specs=[
            pl.BlockSpec(memory_space=pltpu.SMEM),
            pl.BlockSpec(memory_space=pltpu.SMEM),
            pl.BlockSpec((BM, d), lambda t: (jnp.minimum(t, last) // nn, 0)),
            pl.BlockSpec((d, BN), lambda t: (0, jnp.minimum(t, last) % nn)),
        ],
        out_specs=[
            pl.BlockSpec(
                (BM, BN),
                lambda t: (jnp.maximum(t - 1, 0) // nn, jnp.maximum(t - 1, 0) % nn)),
            pl.BlockSpec(
                (BM, BN),
                lambda t: (jnp.maximum(t - 1, 0) // nn, jnp.maximum(t - 1, 0) % nn)),
        ],
        out_shape=[
            jax.ShapeDtypeStruct((M, d), jnp.int32),
            jax.ShapeDtypeStruct((M, d), jnp.bfloat16),
        ],
        scratch_shapes=[pltpu.VMEM((BM, BN), jnp.float32),
                        pltpu.VMEM((BM, BN), jnp.float32)],
    )(boundaries, centroids, x_bf, pi_bf)

    x_hat = pl.pallas_call(
        _unrotate_kernel,
        grid=(nm, nn),
        in_specs=[
            pl.BlockSpec((BM, d), lambda i, j: (i, 0)),
            pl.BlockSpec((BN, d), lambda i, j: (j, 0)),
        ],
        out_specs=pl.BlockSpec((BM, BN), lambda i, j: (i, j)),
        out_shape=jax.ShapeDtypeStruct((M, d), jnp.float32),
        compiler_params=pltpu.CompilerParams(
            dimension_semantics=("parallel", "arbitrary"),
        ),
    )(yhat, pi_bf)

    return (x_hat, idx)


# speedup vs baseline: 1.1793x; 1.1793x over previous
import jax
import jax.numpy as jnp
from jax.experimental import pallas as pl
from jax.experimental.pallas import tpu as pltpu

BM1, BN1 = 512, 1024
BM2, BN2 = 512, 1024


def _quant_matmul_kernel(b_ref, c_ref, x_ref, pi_ref, idx_ref, yhat_ref):
    y = jnp.dot(x_ref[...], pi_ref[...], preferred_element_type=jnp.float32)
    u = jnp.zeros(y.shape, jnp.float32)
    for k in range(b_ref.shape[0]):
        u = u + jnp.where(y > b_ref[k], (c_ref[k + 1] - c_ref[k]) + 16.0, 0.0)
    idx_f = jnp.floor(u * (1.0 / 16.0))
    idx_ref[...] = idx_f.astype(jnp.int32)
    yhat_ref[...] = ((u - 16.0 * idx_f) + c_ref[0]).astype(jnp.bfloat16)


def _unrotate_kernel(yhat_ref, pi_ref, out_ref):
    out_ref[...] = jax.lax.dot_general(
        yhat_ref[...], pi_ref[...],
        dimension_numbers=(((1,), (1,)), ((), ())),
        preferred_element_type=jnp.float32,
    )


def kernel(x, Pi, centroids, boundaries):
    M, d = x.shape

    x_bf = x.astype(jnp.bfloat16)
    pi_bf = Pi.astype(jnp.bfloat16)

    idx, yhat = pl.pallas_call(
        _quant_matmul_kernel,
        grid=(M // BM1, d // BN1),
        in_specs=[
            pl.BlockSpec(memory_space=pltpu.SMEM),
            pl.BlockSpec(memory_space=pltpu.SMEM),
            pl.BlockSpec((BM1, d), lambda i, j: (i, 0)),
            pl.BlockSpec((d, BN1), lambda i, j: (0, j)),
        ],
        out_specs=[
            pl.BlockSpec((BM1, BN1), lambda i, j: (i, j)),
            pl.BlockSpec((BM1, BN1), lambda i, j: (i, j)),
        ],
        out_shape=[
            jax.ShapeDtypeStruct((M, d), jnp.int32),
            jax.ShapeDtypeStruct((M, d), jnp.bfloat16),
        ],
        compiler_params=pltpu.CompilerParams(
            dimension_semantics=("parallel", "arbitrary"),
        ),
    )(boundaries, centroids, x_bf, pi_bf)

    x_hat = pl.pallas_call(
        _unrotate_kernel,
        grid=(M // BM2, d // BN2),
        in_specs=[
            pl.BlockSpec((BM2, d), lambda i, j: (i, 0)),
            pl.BlockSpec((BN2, d), lambda i, j: (j, 0)),
        ],
        out_specs=pl.BlockSpec((BM2, BN2), lambda i, j: (i, j)),
        out_shape=jax.ShapeDtypeStruct((M, d), jnp.float32),
        compiler_params=pltpu.CompilerParams(
            dimension_semantics=("parallel", "arbitrary"),
        ),
    )(yhat, pi_bf)

    return (x_hat, idx)


# speedup vs baseline: 1.1996x; 1.0172x over previous
import jax
import jax.numpy as jnp
from jax.experimental import pallas as pl
from jax.experimental.pallas import tpu as pltpu

BM1, BN1 = 512, 2048
BM2, BN2 = 512, 2048


def _quant_matmul_kernel(b_ref, c_ref, x_ref, pi_ref, idx_ref, yhat_ref):
    y = jnp.dot(x_ref[...], pi_ref[...], preferred_element_type=jnp.float32)
    u = jnp.zeros(y.shape, jnp.float32)
    for k in range(b_ref.shape[0]):
        u = u + jnp.where(y > b_ref[k], (c_ref[k + 1] - c_ref[k]) + 16.0, 0.0)
    idx_f = jnp.floor(u * (1.0 / 16.0))
    idx_ref[...] = idx_f.astype(jnp.int32)
    yhat_ref[...] = ((u - 16.0 * idx_f) + c_ref[0]).astype(jnp.bfloat16)


def _unrotate_kernel(yhat_ref, pi_ref, out_ref):
    out_ref[...] = jax.lax.dot_general(
        yhat_ref[...], pi_ref[...],
        dimension_numbers=(((1,), (1,)), ((), ())),
        preferred_element_type=jnp.float32,
    )


def kernel(x, Pi, centroids, boundaries):
    M, d = x.shape

    x_bf = x.astype(jnp.bfloat16)
    pi_bf = Pi.astype(jnp.bfloat16)

    idx, yhat = pl.pallas_call(
        _quant_matmul_kernel,
        grid=(M // BM1, d // BN1),
        in_specs=[
            pl.BlockSpec(memory_space=pltpu.SMEM),
            pl.BlockSpec(memory_space=pltpu.SMEM),
            pl.BlockSpec((BM1, d), lambda i, j: (i, 0)),
            pl.BlockSpec((d, BN1), lambda i, j: (0, j)),
        ],
        out_specs=[
            pl.BlockSpec((BM1, BN1), lambda i, j: (i, j)),
            pl.BlockSpec((BM1, BN1), lambda i, j: (i, j)),
        ],
        out_shape=[
            jax.ShapeDtypeStruct((M, d), jnp.int32),
            jax.ShapeDtypeStruct((M, d), jnp.bfloat16),
        ],
        compiler_params=pltpu.CompilerParams(
            dimension_semantics=("parallel", "arbitrary"),
        ),
    )(boundaries, centroids, x_bf, pi_bf)

    x_hat = pl.pallas_call(
        _unrotate_kernel,
        grid=(M // BM2, d // BN2),
        in_specs=[
            pl.BlockSpec((BM2, d), lambda i, j: (i, 0)),
            pl.BlockSpec((BN2, d), lambda i, j: (j, 0)),
        ],
        out_specs=pl.BlockSpec((BM2, BN2), lambda i, j: (i, j)),
        out_shape=jax.ShapeDtypeStruct((M, d), jnp.float32),
        compiler_params=pltpu.CompilerParams(
            dimension_semantics=("parallel", "arbitrary"),
        ),
    )(yhat, pi_bf)

    return (x_hat, idx)


# speedup vs baseline: 1.2300x; 1.0253x over previous
import jax
import jax.numpy as jnp
from jax.experimental import pallas as pl
from jax.experimental.pallas import tpu as pltpu

BM1, BN1 = 1024, 1024
BM2, BN2 = 1024, 1024


def _quant_matmul_kernel(b_ref, c_ref, x_ref, pi_ref, idx_ref, yhat_ref):
    y = jnp.dot(x_ref[...], pi_ref[...], preferred_element_type=jnp.float32)
    u = jnp.zeros(y.shape, jnp.float32)
    for k in range(b_ref.shape[0]):
        u = u + jnp.where(y > b_ref[k], (c_ref[k + 1] - c_ref[k]) + 16.0, 0.0)
    idx_f = jnp.floor(u * (1.0 / 16.0))
    idx_ref[...] = idx_f.astype(jnp.int32)
    yhat_ref[...] = ((u - 16.0 * idx_f) + c_ref[0]).astype(jnp.bfloat16)


def _unrotate_kernel(yhat_ref, pi_ref, out_ref):
    out_ref[...] = jax.lax.dot_general(
        yhat_ref[...], pi_ref[...],
        dimension_numbers=(((1,), (1,)), ((), ())),
        preferred_element_type=jnp.float32,
    )


def kernel(x, Pi, centroids, boundaries):
    M, d = x.shape

    x_bf = x.astype(jnp.bfloat16)
    pi_bf = Pi.astype(jnp.bfloat16)

    idx, yhat = pl.pallas_call(
        _quant_matmul_kernel,
        grid=(M // BM1, d // BN1),
        in_specs=[
            pl.BlockSpec(memory_space=pltpu.SMEM),
            pl.BlockSpec(memory_space=pltpu.SMEM),
            pl.BlockSpec((BM1, d), lambda i, j: (i, 0)),
            pl.BlockSpec((d, BN1), lambda i, j: (0, j)),
        ],
        out_specs=[
            pl.BlockSpec((BM1, BN1), lambda i, j: (i, j)),
            pl.BlockSpec((BM1, BN1), lambda i, j: (i, j)),
        ],
        out_shape=[
            jax.ShapeDtypeStruct((M, d), jnp.int32),
            jax.ShapeDtypeStruct((M, d), jnp.bfloat16),
        ],
        compiler_params=pltpu.CompilerParams(
            dimension_semantics=("parallel", "arbitrary"),
        ),
    )(boundaries, centroids, x_bf, pi_bf)

    x_hat = pl.pallas_call(
        _unrotate_kernel,
        grid=(M // BM2, d // BN2),
        in_specs=[
            pl.BlockSpec((BM2, d), lambda i, j: (i, 0)),
            pl.BlockSpec((BN2, d), lambda i, j: (j, 0)),
        ],
        out_specs=pl.BlockSpec((BM2, BN2), lambda i, j: (i, j)),
        out_shape=jax.ShapeDtypeStruct((M, d), jnp.float32),
        compiler_params=pltpu.CompilerParams(
            dimension_semantics=("parallel", "arbitrary"),
        ),
    )(yhat, pi_bf)

    return (x_hat, idx)
